# Initial kernel scaffold; baseline (speedup 1.0000x reference)
import jax
import jax.numpy as jnp
from jax.experimental import pallas as pl


def kernel(y, patches, W_MULT, W_BIAS):
    raise NotImplementedError("write your pallas kernel here")


# speedup vs baseline: 1.9615x; 1.9615x over previous
import functools

import jax
import jax.numpy as jnp
from jax import lax
from jax.experimental import pallas as pl
from jax.experimental.pallas import tpu as pltpu
from jax.experimental.pallas import tpu_sc as plsc

_LANES = 16


def _build_sc_kernel(B, L, C, P, S, NC, NS):
    NW = NC * NS
    PW = P // NW
    CHUNK = 4
    NCH = PW // CHUNK
    WROWS = CHUNK * S
    ROWS = B * WROWS
    GROUPS = PW * B // _LANES
    CVECS = C // _LANES

    @functools.partial(
        pl.kernel,
        mesh=plsc.VectorSubcoreMesh(core_axis_name="c", subcore_axis_name="s"),
        compiler_params=pltpu.CompilerParams(needs_layout_passes=False),
        out_type=jax.ShapeDtypeStruct((P * B,), jnp.float32),
        scratch_types=[
            pltpu.VMEM((NCH, ROWS), jnp.int32),
            pltpu.VMEM((2, WROWS, C), jnp.float32),
            pltpu.VMEM((2, ROWS, C), jnp.float32),
            pltpu.VMEM((PW * B,), jnp.float32),
            pltpu.VMEM((PW * B,), jnp.float32),
            pltpu.SemaphoreType.DMA,
            pltpu.SemaphoreType.DMA,
        ],
    )
    def run(gidx_h, w_h, bias_h, y_h, out_h,
            idx_v, w_buf, rows_buf, bias_v, out_v, sem_a, sem_b):
        cid = lax.axis_index("c")
        sid = lax.axis_index("s")
        wid = sid * NC + cid

        pltpu.sync_copy(gidx_h.at[pl.ds(wid * NCH, NCH)], idx_v)
        pltpu.sync_copy(bias_h.at[pl.ds(wid * PW * B, PW * B)], bias_v)

        sems = (sem_a, sem_b)

        def issue(ch, slot):
            cw = pltpu.async_copy(
                w_h.at[pl.ds((wid * NCH + ch) * WROWS, WROWS)],
                w_buf.at[slot], sems[slot])
            cr = pltpu.async_copy(
                y_h.at[idx_v.at[ch]], rows_buf.at[slot], sems[slot])
            return (cw, cr)

        iota16 = lax.iota(jnp.int32, _LANES)
        zeros = jnp.zeros((_LANES,), jnp.float32)

        pend = issue(0, 0)
        for ch in range(NCH):
            slot = ch % 2
            nxt = issue(ch + 1, 1 - slot) if ch + 1 < NCH else None
            for h in pend:
                h.wait()
            pend = nxt

            def patch_loop(p, pack, slot=slot):
                def ss_loop(ss, accs):
                    row = p * S + ss

                    def cc_loop(cc, accs):
                        off = pl.ds(cc * _LANES, _LANES)
                        w = w_buf[slot, row, off]
                        return tuple(
                            accs[b] + w * rows_buf[slot, b * WROWS + row, off]
                            for b in range(B)
                        )

                    return lax.fori_loop(0, CVECS, cc_loop, accs)

                accs = lax.fori_loop(0, S, ss_loop, (zeros,) * B)
                for b in range(B):
                    tot = lax.broadcast(jnp.sum(accs[b]), (_LANES,))
                    pack = jnp.where(iota16 == p * B + b, tot, pack)
                return pack

            pack = lax.fori_loop(0, CHUNK, patch_loop, zeros)
            o = pack + bias_v[pl.ds(ch * _LANES, _LANES)]
            out_v[pl.ds(ch * _LANES, _LANES)] = jnp.where(o >= 0.0, o, 0.1 * o)

        pltpu.sync_copy(out_v, out_h.at[pl.ds(wid * PW * B, PW * B)])

    return run


def kernel(y, patches, W_MULT, W_BIAS):
    B, L, C = y.shape
    P, S, _ = patches.shape

    info = plsc.get_sparse_core_info()
    NC, NS = info.num_cores, info.num_subcores

    yf = y.reshape(B * L, C)
    w2 = W_MULT.reshape(P * S, C)
    pidx = patches[:, :, 0].astype(jnp.int32).reshape(P // 4, 4 * S)
    gidx = (pidx[:, None, :]
            + (jnp.arange(B, dtype=jnp.int32) * L)[None, :, None]
            ).reshape(P // 4, B * 4 * S)
    bias_rep = jnp.repeat(W_BIAS.reshape(P), B)

    run = _build_sc_kernel(B, L, C, P, S, NC, NS)
    outf = run(gidx, w2, bias_rep, yf)
    return outf.reshape(P, B).T


# speedup vs baseline: 2.3037x; 1.1745x over previous
import functools

import jax
import jax.numpy as jnp
from jax import lax
from jax.experimental import pallas as pl
from jax.experimental.pallas import tpu as pltpu
from jax.experimental.pallas import tpu_sc as plsc

_LANES = 16


def _build_sc_kernel(B, L, C, P, S, NC, NS):
    NW = NC * NS
    PW = P // NW
    CHUNK = 4
    NCH = PW // CHUNK
    WROWS = CHUNK * S
    CVECS = C // _LANES

    @functools.partial(
        pl.kernel,
        mesh=plsc.VectorSubcoreMesh(core_axis_name="c", subcore_axis_name="s"),
        compiler_params=pltpu.CompilerParams(needs_layout_passes=False),
        out_type=jax.ShapeDtypeStruct((P * B,), jnp.float32),
        scratch_types=[
            pltpu.VMEM((NCH, WROWS), jnp.int32),
            pltpu.VMEM((2, CHUNK, S, C), jnp.float32),
            pltpu.VMEM((2, B, WROWS, C), jnp.float32),
            pltpu.VMEM((PW * B,), jnp.float32),
            pltpu.VMEM((PW * B,), jnp.float32),
            pltpu.SemaphoreType.DMA,
            pltpu.SemaphoreType.DMA,
        ],
    )
    def run(gidx_h, w_h, bias_h, y_h, out_h,
            idx_v, w_buf, rows_buf, bias_v, out_v, sem_a, sem_b):
        cid = lax.axis_index("c")
        sid = lax.axis_index("s")
        wid = sid * NC + cid

        pltpu.sync_copy(gidx_h.at[pl.ds(wid * NCH, NCH)], idx_v)
        pltpu.sync_copy(bias_h.at[pl.ds(wid * PW * B, PW * B)], bias_v)

        sems = (sem_a, sem_b)

        def issue(ch, slot):
            hs = [pltpu.async_copy(
                w_h.at[pl.ds((wid * NCH + ch) * CHUNK, CHUNK)],
                w_buf.at[slot], sems[slot])]
            for b in range(B):
                hs.append(pltpu.async_copy(
                    y_h.at[b].at[idx_v.at[ch]], rows_buf.at[slot, b],
                    sems[slot]))
            return hs

        iota16 = lax.iota(jnp.int32, _LANES)
        zeros = jnp.zeros((_LANES,), jnp.float32)

        pend = issue(0, 0)
        for ch in range(NCH):
            slot = ch % 2
            nxt = issue(ch + 1, 1 - slot) if ch + 1 < NCH else None
            for h in pend:
                h.wait()
            pend = nxt

            def patch_loop(p, pack, slot=slot):
                def ss_loop(ss, accs):
                    row = p * S + ss

                    def cc_loop(cc, accs):
                        off = pl.ds(cc * _LANES, _LANES)
                        w = w_buf[slot, p, ss, off]
                        return tuple(
                            accs[b] + w * rows_buf[slot, b, row, off]
                            for b in range(B)
                        )

                    return lax.fori_loop(0, CVECS, cc_loop, accs)

                accs = lax.fori_loop(0, S, ss_loop, (zeros,) * B)
                for b in range(B):
                    tot = lax.broadcast(jnp.sum(accs[b]), (_LANES,))
                    pack = jnp.where(iota16 == p * B + b, tot, pack)
                return pack

            pack = lax.fori_loop(0, CHUNK, patch_loop, zeros)
            o = pack + bias_v[pl.ds(ch * _LANES, _LANES)]
            out_v[pl.ds(ch * _LANES, _LANES)] = jnp.where(o >= 0.0, o, 0.1 * o)

        pltpu.sync_copy(out_v, out_h.at[pl.ds(wid * PW * B, PW * B)])

    return run


def kernel(y, patches, W_MULT, W_BIAS):
    B, L, C = y.shape
    P, S, _ = patches.shape

    info = plsc.get_sparse_core_info()
    NC, NS = info.num_cores, info.num_subcores

    w3 = W_MULT.reshape(P, S, C)
    gidx = patches[:, :, 0].astype(jnp.int32).reshape(P // 4, 4 * S)
    bias_rep = jnp.repeat(W_BIAS.reshape(P), B)

    run = _build_sc_kernel(B, L, C, P, S, NC, NS)
    outf = run(gidx, w3, bias_rep, y)
    return outf.reshape(P, B).T
